# baseline (speedup 1.0000x reference)
import functools

import jax
import jax.numpy as jnp
from jax import lax
from jax.experimental import pallas as pl
from jax.experimental.pallas import tpu as pltpu
from jax.experimental.pallas import tpu_sc as plsc

N = 10000
E = 320000
IN_CH = 128
HID = 16
OUT_CH = 64

NC = 2
NS = 16
NW = NC * NS
CHUNK = 128
NCHUNKS = -(-E // CHUNK)
NCHUNKS_PAD = -(-NCHUNKS // (NW * 8)) * (NW * 8)
CH_PER_W = NCHUNKS_PAD // NW
E_PAD = NCHUNKS_PAD * CHUNK
NA = N + 8
ROWS_PER_TILE = 624
ROWS_REM = N - NS * ROWS_PER_TILE
DEG_F = 8
DEG_K = 16
AGG_K = 4
AGG_NH = 2
AGG_HCH = CH_PER_W // AGG_NH


def _mesh():
    return plsc.VectorSubcoreMesh(core_axis_name="c", subcore_axis_name="s")


def _worker_ids():
    c = lax.axis_index("c")
    s = lax.axis_index("s")
    w = c * NS + s
    return c, s, w


def _tile_rowwise_copy(s, src_ref, dst_ref):
    pltpu.sync_copy(src_ref.at[pl.ds(s * ROWS_PER_TILE, ROWS_PER_TILE)],
                    dst_ref.at[pl.ds(s * ROWS_PER_TILE, ROWS_PER_TILE)])

    @pl.when(s == NS - 1)
    def _():
        pltpu.sync_copy(src_ref.at[pl.ds(NS * ROWS_PER_TILE, ROWS_REM)],
                        dst_ref.at[pl.ds(NS * ROWS_PER_TILE, ROWS_REM)])


def _sc_deg(dst2, ones_rows, zeros_acc):
    @functools.partial(
        pl.kernel,
        out_type=jax.ShapeDtypeStruct((NC, N, DEG_F), jnp.float32),
        mesh=_mesh(),
        scratch_types=[
            pltpu.VMEM((CH_PER_W, CHUNK), jnp.int32),
            pltpu.VMEM((CHUNK, DEG_F), jnp.float32),
            pltpu.VMEM_SHARED((NA, DEG_F), jnp.float32),
            pltpu.SemaphoreType.DMA,
        ],
        compiler_params=pltpu.CompilerParams(use_tc_tiling_on_sc=False),
    )
    def k(dst_hbm, ones_hbm, z_hbm, out_hbm, didx, rows, acc, dsem):
        c, s, w = _worker_ids()
        base = w * CH_PER_W
        _tile_rowwise_copy(s, z_hbm, acc)

        @pl.when(s == NS - 1)
        def _():
            pltpu.sync_copy(z_hbm.at[pl.ds(N, NA - N)], acc.at[pl.ds(N, NA - N)])

        pltpu.sync_copy(ones_hbm, rows)
        pltpu.sync_copy(dst_hbm.at[pl.ds(base, CH_PER_W)], didx)
        plsc.subcore_barrier()

        def body(r, carry):
            for b in range(DEG_K):
                pltpu.async_copy(rows, acc.at[didx.at[r * DEG_K + b]], dsem,
                                 add=True)
            for b in range(DEG_K):
                pltpu.make_async_copy(rows, acc.at[didx.at[0]], dsem).wait()
            return carry

        lax.fori_loop(0, CH_PER_W // DEG_K, body, 0)

        plsc.subcore_barrier()
        _tile_rowwise_copy(s, acc, out_hbm.at[c])

    return k(dst2, ones_rows, zeros_acc)


def _sc_agg(g, src2, dst2, zeros_acc, F):
    @functools.partial(
        pl.kernel,
        out_type=jax.ShapeDtypeStruct((NC, N, F), jnp.float32),
        mesh=_mesh(),
        scratch_types=[
            pltpu.VMEM((AGG_HCH, CHUNK), jnp.int32),
            pltpu.VMEM((AGG_HCH, CHUNK), jnp.int32),
            pltpu.VMEM((AGG_K, CHUNK, F), jnp.float32),
            pltpu.VMEM_SHARED((N, F), jnp.float32),
            pltpu.VMEM_SHARED((NA, F), jnp.float32),
            pltpu.SemaphoreType.DMA((AGG_K,)),
            pltpu.SemaphoreType.DMA((AGG_K,)),
        ],
        compiler_params=pltpu.CompilerParams(use_tc_tiling_on_sc=False),
    )
    def k(g_hbm, src_hbm, dst_hbm, z_hbm, out_hbm, sidx, didx, rows, tbl, acc,
          gsem, ssem):
        c, s, w = _worker_ids()
        base = w * CH_PER_W
        _tile_rowwise_copy(s, z_hbm, acc)
        _tile_rowwise_copy(s, g_hbm, tbl)

        @pl.when(s == NS - 1)
        def _():
            pltpu.sync_copy(z_hbm.at[pl.ds(N, NA - N)], acc.at[pl.ds(N, NA - N)])

        plsc.subcore_barrier()

        n_groups = AGG_HCH // AGG_K

        def body(r, carry):
            for b in range(AGG_K):
                j = r * AGG_K + b
                pltpu.make_async_copy(
                    tbl.at[sidx.at[j]], rows.at[b], gsem.at[b]).wait()
                pltpu.async_copy(
                    rows.at[b], acc.at[didx.at[j]], ssem.at[b], add=True)

            @pl.when(r < n_groups - 1)
            def _():
                for b in range(AGG_K):
                    j = r * AGG_K + b
                    pltpu.make_async_copy(
                        rows.at[b], acc.at[didx.at[j]], ssem.at[b]).wait()
                    pltpu.async_copy(
                        tbl.at[sidx.at[j + AGG_K]], rows.at[b], gsem.at[b])

            return carry

        for h in range(AGG_NH):
            pltpu.sync_copy(src_hbm.at[pl.ds(base + h * AGG_HCH, AGG_HCH)],
                            sidx)
            pltpu.sync_copy(dst_hbm.at[pl.ds(base + h * AGG_HCH, AGG_HCH)],
                            didx)
            for b in range(AGG_K):
                pltpu.async_copy(tbl.at[sidx.at[b]], rows.at[b], gsem.at[b])
            lax.fori_loop(0, n_groups, body, 0)
            for b in range(AGG_K):
                pltpu.make_async_copy(
                    rows.at[b], acc.at[didx.at[b]], ssem.at[b]).wait()

        plsc.subcore_barrier()
        _tile_rowwise_copy(s, acc, out_hbm.at[c])

    return k(g, src2, dst2, zeros_acc)


def _tc1a_body(x_ref, w1_ref, h1_ref):
    h1_ref[...] = jnp.dot(x_ref[...], w1_ref[...],
                          preferred_element_type=jnp.float32)


def _tc1b_body(h1_ref, degp_ref, g1_ref, dinv_ref):
    d = 1.0 + degp_ref[0] + degp_ref[1]
    dinv = lax.rsqrt(d)
    g1_ref[...] = h1_ref[...] * dinv[:, 0:1]
    dinv_ref[...] = dinv


def _tc2_body(p1_ref, g1_ref, dinv_ref, w2_ref, b1_ref, g2_ref):
    dinv = dinv_ref[...][:, 0:1]
    agg = p1_ref[0] + p1_ref[1] + g1_ref[...]
    h1 = jnp.maximum(dinv * agg + b1_ref[...], 0.0)
    h2 = jnp.dot(h1, w2_ref[...], preferred_element_type=jnp.float32)
    g2_ref[...] = h2 * dinv


def _tc3_body(p2_ref, g2_ref, dinv_ref, b2_ref, lw1_ref, lb1_ref, lw2_ref,
              lb2_ref, out_ref):
    dinv = dinv_ref[...][:, 0:1]
    agg = p2_ref[0] + p2_ref[1] + g2_ref[...]
    h2 = jnp.maximum(dinv * agg + b2_ref[...], 0.0)
    u = jnp.maximum(
        jnp.dot(h2, lw1_ref[...], preferred_element_type=jnp.float32)
        + lb1_ref[...], 0.0)
    out_ref[...] = (
        jnp.dot(u, lw2_ref[...], preferred_element_type=jnp.float32)
        + lb2_ref[...])


def kernel(x, edge_index, W1, b1, W2, b2, LW1, Lb1, LW2, Lb2):
    src = edge_index[0].astype(jnp.int32)
    dst = edge_index[1].astype(jnp.int32)
    pad = E_PAD - E
    src2 = jnp.concatenate([src, jnp.zeros((pad,), jnp.int32)])
    src2 = src2.reshape(NCHUNKS_PAD, CHUNK)
    dst2 = jnp.concatenate([dst, jnp.full((pad,), N, jnp.int32)])
    dst2 = dst2.reshape(NCHUNKS_PAD, CHUNK)
    ones_rows = jnp.ones((CHUNK, DEG_F), jnp.float32)
    zdeg = jnp.zeros((NA, DEG_F), jnp.float32)
    z16 = jnp.zeros((NA, HID), jnp.float32)
    z64 = jnp.zeros((NA, OUT_CH), jnp.float32)

    h1 = pl.pallas_call(
        _tc1a_body,
        out_shape=jax.ShapeDtypeStruct((N, HID), jnp.float32),
    )(x, W1)

    degp = _sc_deg(dst2, ones_rows, zdeg)

    g1, dinv = pl.pallas_call(
        _tc1b_body,
        out_shape=[jax.ShapeDtypeStruct((N, HID), jnp.float32),
                   jax.ShapeDtypeStruct((N, DEG_F), jnp.float32)],
    )(h1, degp)

    p1 = _sc_agg(g1, src2, dst2, z16, HID)

    g2 = pl.pallas_call(
        _tc2_body,
        out_shape=jax.ShapeDtypeStruct((N, OUT_CH), jnp.float32),
    )(p1, g1, dinv, W2, b1.reshape(1, HID))

    p2 = _sc_agg(g2, src2, dst2, z64, OUT_CH)

    out = pl.pallas_call(
        _tc3_body,
        out_shape=jax.ShapeDtypeStruct((N, 1), jnp.float32),
    )(p2, g2, dinv, b2.reshape(1, OUT_CH), LW1, Lb1.reshape(1, OUT_CH),
      LW2, Lb2.reshape(1, 1))

    return out.reshape(N)


# speedup vs baseline: 1.0584x; 1.0584x over previous
import functools

import jax
import jax.numpy as jnp
from jax import lax
from jax.experimental import pallas as pl
from jax.experimental.pallas import tpu as pltpu
from jax.experimental.pallas import tpu_sc as plsc

N = 10000
E = 320000
IN_CH = 128
HID = 16
OUT_CH = 64

NC = 2
NS = 16
NW = NC * NS
CHUNK = 128
NCHUNKS = -(-E // CHUNK)
NCHUNKS_PAD = -(-NCHUNKS // (NW * 8)) * (NW * 8)
CH_PER_W = NCHUNKS_PAD // NW
E_PAD = NCHUNKS_PAD * CHUNK
NA = N + 8
ROWS_PER_TILE = 624
ROWS_REM = N - NS * ROWS_PER_TILE
DEG_F = 8
DEG_K = 16
AGG_K = 2
AGG_NH = 1
AGG_HCH = CH_PER_W // AGG_NH


def _mesh():
    return plsc.VectorSubcoreMesh(core_axis_name="c", subcore_axis_name="s")


def _worker_ids():
    c = lax.axis_index("c")
    s = lax.axis_index("s")
    w = c * NS + s
    return c, s, w


def _tile_rowwise_copy(s, src_ref, dst_ref):
    pltpu.sync_copy(src_ref.at[pl.ds(s * ROWS_PER_TILE, ROWS_PER_TILE)],
                    dst_ref.at[pl.ds(s * ROWS_PER_TILE, ROWS_PER_TILE)])

    @pl.when(s == NS - 1)
    def _():
        pltpu.sync_copy(src_ref.at[pl.ds(NS * ROWS_PER_TILE, ROWS_REM)],
                        dst_ref.at[pl.ds(NS * ROWS_PER_TILE, ROWS_REM)])


def _sc_deg(dst2, ones_rows, zeros_acc):
    @functools.partial(
        pl.kernel,
        out_type=jax.ShapeDtypeStruct((NC, N, DEG_F), jnp.float32),
        mesh=_mesh(),
        scratch_types=[
            pltpu.VMEM((CH_PER_W, CHUNK), jnp.int32),
            pltpu.VMEM((CHUNK, DEG_F), jnp.float32),
            pltpu.VMEM_SHARED((NA, DEG_F), jnp.float32),
            pltpu.SemaphoreType.DMA,
        ],
        compiler_params=pltpu.CompilerParams(use_tc_tiling_on_sc=False),
    )
    def k(dst_hbm, ones_hbm, z_hbm, out_hbm, didx, rows, acc, dsem):
        c, s, w = _worker_ids()
        base = w * CH_PER_W
        _tile_rowwise_copy(s, z_hbm, acc)

        @pl.when(s == NS - 1)
        def _():
            pltpu.sync_copy(z_hbm.at[pl.ds(N, NA - N)], acc.at[pl.ds(N, NA - N)])

        pltpu.sync_copy(ones_hbm, rows)
        pltpu.sync_copy(dst_hbm.at[pl.ds(base, CH_PER_W)], didx)
        plsc.subcore_barrier()

        def body(r, carry):
            for b in range(DEG_K):
                pltpu.async_copy(rows, acc.at[didx.at[r * DEG_K + b]], dsem,
                                 add=True)
            for b in range(DEG_K):
                pltpu.make_async_copy(rows, acc.at[didx.at[0]], dsem).wait()
            return carry

        lax.fori_loop(0, CH_PER_W // DEG_K, body, 0)

        plsc.subcore_barrier()
        _tile_rowwise_copy(s, acc, out_hbm.at[c])

    return k(dst2, ones_rows, zeros_acc)


def _sc_agg(g, src2, dst2, zeros_acc, F):
    @functools.partial(
        pl.kernel,
        out_type=jax.ShapeDtypeStruct((NC, N, F), jnp.float32),
        mesh=_mesh(),
        scratch_types=[
            pltpu.VMEM((AGG_HCH, CHUNK), jnp.int32),
            pltpu.VMEM((AGG_HCH, CHUNK), jnp.int32),
            pltpu.VMEM((AGG_K, CHUNK, F), jnp.float32),
            pltpu.VMEM_SHARED((N, F), jnp.float32),
            pltpu.VMEM_SHARED((NA, F), jnp.float32),
            pltpu.SemaphoreType.DMA((AGG_K,)),
            pltpu.SemaphoreType.DMA((AGG_K,)),
        ],
        compiler_params=pltpu.CompilerParams(use_tc_tiling_on_sc=False),
    )
    def k(g_hbm, src_hbm, dst_hbm, z_hbm, out_hbm, sidx, didx, rows, tbl, acc,
          gsem, ssem):
        c, s, w = _worker_ids()
        base = w * CH_PER_W
        rsl = pl.ds(s * ROWS_PER_TILE, ROWS_PER_TILE)
        d0 = pltpu.async_copy(z_hbm.at[rsl], acc.at[rsl], ssem.at[0])
        d1 = pltpu.async_copy(g_hbm.at[rsl], tbl.at[rsl], ssem.at[1])
        d2 = pltpu.async_copy(src_hbm.at[pl.ds(base, CH_PER_W)], sidx,
                              gsem.at[0])
        d3 = pltpu.async_copy(dst_hbm.at[pl.ds(base, CH_PER_W)], didx,
                              gsem.at[1])

        @pl.when(s == NS - 1)
        def _():
            tsl = pl.ds(NS * ROWS_PER_TILE, ROWS_REM)
            pltpu.sync_copy(z_hbm.at[tsl], acc.at[tsl])
            pltpu.sync_copy(g_hbm.at[tsl], tbl.at[tsl])
            pltpu.sync_copy(z_hbm.at[pl.ds(N, NA - N)], acc.at[pl.ds(N, NA - N)])

        d0.wait()
        d1.wait()
        d2.wait()
        d3.wait()
        plsc.subcore_barrier()

        n_groups = CH_PER_W // AGG_K

        def body(r, carry):
            for b in range(AGG_K):
                j = r * AGG_K + b
                pltpu.make_async_copy(
                    tbl.at[sidx.at[j]], rows.at[b], gsem.at[b]).wait()
                pltpu.async_copy(
                    rows.at[b], acc.at[didx.at[j]], ssem.at[b], add=True)

            @pl.when(r < n_groups - 1)
            def _():
                for b in range(AGG_K):
                    j = r * AGG_K + b
                    pltpu.make_async_copy(
                        rows.at[b], acc.at[didx.at[j]], ssem.at[b]).wait()
                    pltpu.async_copy(
                        tbl.at[sidx.at[j + AGG_K]], rows.at[b], gsem.at[b])

            return carry

        for b in range(AGG_K):
            pltpu.async_copy(tbl.at[sidx.at[b]], rows.at[b], gsem.at[b])
        lax.fori_loop(0, n_groups, body, 0)
        for b in range(AGG_K):
            pltpu.make_async_copy(
                rows.at[b], acc.at[didx.at[b]], ssem.at[b]).wait()

        plsc.subcore_barrier()
        _tile_rowwise_copy(s, acc, out_hbm.at[c])

    return k(g, src2, dst2, zeros_acc)


def _tc1_body(x_ref, w1_ref, degp_ref, g1_ref, dinv_ref):
    d = 1.0 + degp_ref[0] + degp_ref[1]
    dinv = lax.rsqrt(d)
    h = jnp.dot(x_ref[...], w1_ref[...], preferred_element_type=jnp.float32)
    g1_ref[...] = h * dinv[:, 0:1]
    dinv_ref[...] = dinv


def _tc2_body(p1_ref, g1_ref, dinv_ref, w2_ref, b1_ref, g2_ref):
    dinv = dinv_ref[...][:, 0:1]
    agg = p1_ref[0] + p1_ref[1] + g1_ref[...]
    h1 = jnp.maximum(dinv * agg + b1_ref[...], 0.0)
    h2 = jnp.dot(h1, w2_ref[...], preferred_element_type=jnp.float32)
    g2_ref[...] = h2 * dinv


def _tc3_body(p2_ref, g2_ref, dinv_ref, b2_ref, lw1_ref, lb1_ref, lw2_ref,
              lb2_ref, out_ref):
    dinv = dinv_ref[...][:, 0:1]
    agg = p2_ref[0] + p2_ref[1] + g2_ref[...]
    h2 = jnp.maximum(dinv * agg + b2_ref[...], 0.0)
    u = jnp.maximum(
        jnp.dot(h2, lw1_ref[...], preferred_element_type=jnp.float32)
        + lb1_ref[...], 0.0)
    out_ref[...] = (
        jnp.dot(u, lw2_ref[...], preferred_element_type=jnp.float32)
        + lb2_ref[...])


def kernel(x, edge_index, W1, b1, W2, b2, LW1, Lb1, LW2, Lb2):
    src = edge_index[0].astype(jnp.int32)
    dst = edge_index[1].astype(jnp.int32)
    pad = E_PAD - E
    src2 = jnp.concatenate([src, jnp.zeros((pad,), jnp.int32)])
    src2 = src2.reshape(NCHUNKS_PAD, CHUNK)
    dst2 = jnp.concatenate([dst, jnp.full((pad,), N, jnp.int32)])
    dst2 = dst2.reshape(NCHUNKS_PAD, CHUNK)
    ones_rows = jnp.ones((CHUNK, DEG_F), jnp.float32)
    zdeg = jnp.zeros((NA, DEG_F), jnp.float32)
    z16 = jnp.zeros((NA, HID), jnp.float32)
    z64 = jnp.zeros((NA, OUT_CH), jnp.float32)

    degp = _sc_deg(dst2, ones_rows, zdeg)

    g1, dinv = pl.pallas_call(
        _tc1_body,
        out_shape=[jax.ShapeDtypeStruct((N, HID), jnp.float32),
                   jax.ShapeDtypeStruct((N, DEG_F), jnp.float32)],
    )(x, W1, degp)

    p1 = _sc_agg(g1, src2, dst2, z16, HID)

    g2 = pl.pallas_call(
        _tc2_body,
        out_shape=jax.ShapeDtypeStruct((N, OUT_CH), jnp.float32),
    )(p1, g1, dinv, W2, b1.reshape(1, HID))

    p2 = _sc_agg(g2, src2, dst2, z64, OUT_CH)

    out = pl.pallas_call(
        _tc3_body,
        out_shape=jax.ShapeDtypeStruct((N, 1), jnp.float32),
    )(p2, g2, dinv, b2.reshape(1, OUT_CH), LW1, Lb1.reshape(1, OUT_CH),
      LW2, Lb2.reshape(1, 1))

    return out.reshape(N)


# speedup vs baseline: 1.0609x; 1.0024x over previous
import functools

import jax
import jax.numpy as jnp
from jax import lax
from jax.experimental import pallas as pl
from jax.experimental.pallas import tpu as pltpu
from jax.experimental.pallas import tpu_sc as plsc

N = 10000
E = 320000
IN_CH = 128
HID = 16
OUT_CH = 64

NC = 2
NS = 16
NW = NC * NS
CHUNK = 128
NCHUNKS = -(-E // CHUNK)
NCHUNKS_PAD = -(-NCHUNKS // (NW * 8)) * (NW * 8)
CH_PER_W = NCHUNKS_PAD // NW
E_PAD = NCHUNKS_PAD * CHUNK
NA = N + 8
ROWS_PER_TILE = 624
ROWS_REM = N - NS * ROWS_PER_TILE
DEG_F = 8
DEG_K = 16
AGG_K = 2
AGG_NH = 1
AGG_HCH = CH_PER_W // AGG_NH


def _mesh():
    return plsc.VectorSubcoreMesh(core_axis_name="c", subcore_axis_name="s")


def _worker_ids():
    c = lax.axis_index("c")
    s = lax.axis_index("s")
    w = c * NS + s
    return c, s, w


def _tile_rowwise_copy(s, src_ref, dst_ref):
    pltpu.sync_copy(src_ref.at[pl.ds(s * ROWS_PER_TILE, ROWS_PER_TILE)],
                    dst_ref.at[pl.ds(s * ROWS_PER_TILE, ROWS_PER_TILE)])

    @pl.when(s == NS - 1)
    def _():
        pltpu.sync_copy(src_ref.at[pl.ds(NS * ROWS_PER_TILE, ROWS_REM)],
                        dst_ref.at[pl.ds(NS * ROWS_PER_TILE, ROWS_REM)])


def _sc_deg(dst2, ones_rows, zeros_acc):
    @functools.partial(
        pl.kernel,
        out_type=jax.ShapeDtypeStruct((NC, N, DEG_F), jnp.float32),
        mesh=_mesh(),
        scratch_types=[
            pltpu.VMEM((CH_PER_W, CHUNK), jnp.int32),
            pltpu.VMEM((CHUNK, DEG_F), jnp.float32),
            pltpu.VMEM_SHARED((NA, DEG_F), jnp.float32),
            pltpu.SemaphoreType.DMA,
            pltpu.SemaphoreType.DMA((2,)),
        ],
        compiler_params=pltpu.CompilerParams(use_tc_tiling_on_sc=False),
    )
    def k(dst_hbm, ones_hbm, z_hbm, out_hbm, didx, rows, acc, dsem, psem):
        c, s, w = _worker_ids()
        base = w * CH_PER_W
        rsl = pl.ds(s * ROWS_PER_TILE, ROWS_PER_TILE)
        d0 = pltpu.async_copy(z_hbm.at[rsl], acc.at[rsl], psem.at[0])
        d1 = pltpu.async_copy(dst_hbm.at[pl.ds(base, CH_PER_W)], didx,
                              psem.at[1])
        pltpu.sync_copy(ones_hbm, rows)

        @pl.when(s == NS - 1)
        def _():
            tsl = pl.ds(NS * ROWS_PER_TILE, ROWS_REM)
            pltpu.sync_copy(z_hbm.at[tsl], acc.at[tsl])
            pltpu.sync_copy(z_hbm.at[pl.ds(N, NA - N)], acc.at[pl.ds(N, NA - N)])

        d0.wait()
        d1.wait()
        plsc.subcore_barrier()

        def body(r, carry):
            for b in range(DEG_K):
                pltpu.async_copy(rows, acc.at[didx.at[r * DEG_K + b]], dsem,
                                 add=True)
            for b in range(DEG_K):
                pltpu.make_async_copy(rows, acc.at[didx.at[0]], dsem).wait()
            return carry

        lax.fori_loop(0, CH_PER_W // DEG_K, body, 0)

        plsc.subcore_barrier()
        _tile_rowwise_copy(s, acc, out_hbm.at[c])

    return k(dst2, ones_rows, zeros_acc)


def _sc_agg(g, src2, dst2, zeros_acc, F):
    @functools.partial(
        pl.kernel,
        out_type=jax.ShapeDtypeStruct((NC, N, F), jnp.float32),
        mesh=_mesh(),
        scratch_types=[
            pltpu.VMEM((AGG_HCH, CHUNK), jnp.int32),
            pltpu.VMEM((AGG_HCH, CHUNK), jnp.int32),
            pltpu.VMEM((AGG_K, CHUNK, F), jnp.float32),
            pltpu.VMEM_SHARED((N, F), jnp.float32),
            pltpu.VMEM_SHARED((NA, F), jnp.float32),
            pltpu.SemaphoreType.DMA((AGG_K,)),
            pltpu.SemaphoreType.DMA((AGG_K,)),
        ],
        compiler_params=pltpu.CompilerParams(use_tc_tiling_on_sc=False),
    )
    def k(g_hbm, src_hbm, dst_hbm, z_hbm, out_hbm, sidx, didx, rows, tbl, acc,
          gsem, ssem):
        c, s, w = _worker_ids()
        base = w * CH_PER_W
        rsl = pl.ds(s * ROWS_PER_TILE, ROWS_PER_TILE)
        d0 = pltpu.async_copy(z_hbm.at[rsl], acc.at[rsl], ssem.at[0])
        d1 = pltpu.async_copy(g_hbm.at[rsl], tbl.at[rsl], ssem.at[1])
        d2 = pltpu.async_copy(src_hbm.at[pl.ds(base, CH_PER_W)], sidx,
                              gsem.at[0])
        d3 = pltpu.async_copy(dst_hbm.at[pl.ds(base, CH_PER_W)], didx,
                              gsem.at[1])

        @pl.when(s == NS - 1)
        def _():
            tsl = pl.ds(NS * ROWS_PER_TILE, ROWS_REM)
            pltpu.sync_copy(z_hbm.at[tsl], acc.at[tsl])
            pltpu.sync_copy(g_hbm.at[tsl], tbl.at[tsl])
            pltpu.sync_copy(z_hbm.at[pl.ds(N, NA - N)], acc.at[pl.ds(N, NA - N)])

        d0.wait()
        d1.wait()
        d2.wait()
        d3.wait()
        plsc.subcore_barrier()

        n_groups = CH_PER_W // AGG_K

        def body(r, carry):
            for b in range(AGG_K):
                j = r * AGG_K + b
                pltpu.make_async_copy(
                    tbl.at[sidx.at[j]], rows.at[b], gsem.at[b]).wait()
                pltpu.async_copy(
                    rows.at[b], acc.at[didx.at[j]], ssem.at[b], add=True)

            @pl.when(r < n_groups - 1)
            def _():
                for b in range(AGG_K):
                    j = r * AGG_K + b
                    pltpu.make_async_copy(
                        rows.at[b], acc.at[didx.at[j]], ssem.at[b]).wait()
                    pltpu.async_copy(
                        tbl.at[sidx.at[j + AGG_K]], rows.at[b], gsem.at[b])

            return carry

        for b in range(AGG_K):
            pltpu.async_copy(tbl.at[sidx.at[b]], rows.at[b], gsem.at[b])
        lax.fori_loop(0, n_groups, body, 0)
        for b in range(AGG_K):
            pltpu.make_async_copy(
                rows.at[b], acc.at[didx.at[b]], ssem.at[b]).wait()

        plsc.subcore_barrier()
        _tile_rowwise_copy(s, acc, out_hbm.at[c])

    return k(g, src2, dst2, zeros_acc)


def _tc1_body(x_ref, w1_ref, degp_ref, g1_ref, dinv_ref):
    d = 1.0 + degp_ref[0] + degp_ref[1]
    dinv = lax.rsqrt(d)
    h = jnp.dot(x_ref[...], w1_ref[...], preferred_element_type=jnp.float32)
    g1_ref[...] = h * dinv[:, 0:1]
    dinv_ref[...] = dinv


def _tc2_body(p1_ref, g1_ref, dinv_ref, w2_ref, b1_ref, g2_ref):
    dinv = dinv_ref[...][:, 0:1]
    agg = p1_ref[0] + p1_ref[1] + g1_ref[...]
    h1 = jnp.maximum(dinv * agg + b1_ref[...], 0.0)
    h2 = jnp.dot(h1, w2_ref[...], preferred_element_type=jnp.float32)
    g2_ref[...] = h2 * dinv


def _tc3_body(p2_ref, g2_ref, dinv_ref, b2_ref, lw1_ref, lb1_ref, lw2_ref,
              lb2_ref, out_ref):
    dinv = dinv_ref[...][:, 0:1]
    agg = p2_ref[0] + p2_ref[1] + g2_ref[...]
    h2 = jnp.maximum(dinv * agg + b2_ref[...], 0.0)
    u = jnp.maximum(
        jnp.dot(h2, lw1_ref[...], preferred_element_type=jnp.float32)
        + lb1_ref[...], 0.0)
    out_ref[...] = (
        jnp.dot(u, lw2_ref[...], preferred_element_type=jnp.float32)
        + lb2_ref[...])


def kernel(x, edge_index, W1, b1, W2, b2, LW1, Lb1, LW2, Lb2):
    src = edge_index[0].astype(jnp.int32)
    dst = edge_index[1].astype(jnp.int32)
    pad = E_PAD - E
    src2 = jnp.concatenate([src, jnp.zeros((pad,), jnp.int32)])
    src2 = src2.reshape(NCHUNKS_PAD, CHUNK)
    dst2 = jnp.concatenate([dst, jnp.full((pad,), N, jnp.int32)])
    dst2 = dst2.reshape(NCHUNKS_PAD, CHUNK)
    ones_rows = jnp.ones((CHUNK, DEG_F), jnp.float32)
    zdeg = jnp.zeros((NA, DEG_F), jnp.float32)
    z16 = jnp.zeros((NA, HID), jnp.float32)
    z64 = jnp.zeros((NA, OUT_CH), jnp.float32)

    degp = _sc_deg(dst2, ones_rows, zdeg)

    g1, dinv = pl.pallas_call(
        _tc1_body,
        out_shape=[jax.ShapeDtypeStruct((N, HID), jnp.float32),
                   jax.ShapeDtypeStruct((N, DEG_F), jnp.float32)],
    )(x, W1, degp)

    p1 = _sc_agg(g1, src2, dst2, z16, HID)

    g2 = pl.pallas_call(
        _tc2_body,
        out_shape=jax.ShapeDtypeStruct((N, OUT_CH), jnp.float32),
    )(p1, g1, dinv, W2, b1.reshape(1, HID))

    p2 = _sc_agg(g2, src2, dst2, z64, OUT_CH)

    out = pl.pallas_call(
        _tc3_body,
        out_shape=jax.ShapeDtypeStruct((N, 1), jnp.float32),
    )(p2, g2, dinv, b2.reshape(1, OUT_CH), LW1, Lb1.reshape(1, OUT_CH),
      LW2, Lb2.reshape(1, 1))

    return out.reshape(N)


# speedup vs baseline: 1.0646x; 1.0034x over previous
import functools

import jax
import jax.numpy as jnp
from jax import lax
from jax.experimental import pallas as pl
from jax.experimental.pallas import tpu as pltpu
from jax.experimental.pallas import tpu_sc as plsc

N = 10000
E = 320000
IN_CH = 128
HID = 16
OUT_CH = 64

NC = 2
NS = 16
NW = NC * NS
CHUNK = 128
NCHUNKS = -(-E // CHUNK)
NCHUNKS_PAD = -(-NCHUNKS // (NW * 8)) * (NW * 8)
CH_PER_W = NCHUNKS_PAD // NW
E_PAD = NCHUNKS_PAD * CHUNK
NA = N + 8
ROWS_PER_TILE = 624
ROWS_REM = N - NS * ROWS_PER_TILE
DEG_F = 8
DEG_K = 16
AGG_K = 2
AGG_NH = 1
AGG_HCH = CH_PER_W // AGG_NH


def _mesh():
    return plsc.VectorSubcoreMesh(core_axis_name="c", subcore_axis_name="s")


def _worker_ids():
    c = lax.axis_index("c")
    s = lax.axis_index("s")
    w = c * NS + s
    return c, s, w


def _tile_rowwise_copy(s, src_ref, dst_ref):
    pltpu.sync_copy(src_ref.at[pl.ds(s * ROWS_PER_TILE, ROWS_PER_TILE)],
                    dst_ref.at[pl.ds(s * ROWS_PER_TILE, ROWS_PER_TILE)])

    @pl.when(s == NS - 1)
    def _():
        pltpu.sync_copy(src_ref.at[pl.ds(NS * ROWS_PER_TILE, ROWS_REM)],
                        dst_ref.at[pl.ds(NS * ROWS_PER_TILE, ROWS_REM)])


def _sc_deg(dst2, ones_rows, zeros_acc):
    @functools.partial(
        pl.kernel,
        out_type=jax.ShapeDtypeStruct((NC, N, DEG_F), jnp.float32),
        mesh=_mesh(),
        scratch_types=[
            pltpu.VMEM((CH_PER_W, CHUNK), jnp.int32),
            pltpu.VMEM((CHUNK, DEG_F), jnp.float32),
            pltpu.VMEM_SHARED((NA, DEG_F), jnp.float32),
            pltpu.SemaphoreType.DMA,
            pltpu.SemaphoreType.DMA((2,)),
        ],
        compiler_params=pltpu.CompilerParams(use_tc_tiling_on_sc=False),
    )
    def k(dst_hbm, ones_hbm, z_hbm, out_hbm, didx, rows, acc, dsem, psem):
        c, s, w = _worker_ids()
        base = w * CH_PER_W
        rsl = pl.ds(s * ROWS_PER_TILE, ROWS_PER_TILE)
        d0 = pltpu.async_copy(z_hbm.at[rsl], acc.at[rsl], psem.at[0])
        d1 = pltpu.async_copy(dst_hbm.at[pl.ds(base, CH_PER_W)], didx,
                              psem.at[1])
        pltpu.sync_copy(ones_hbm, rows)

        @pl.when(s == NS - 1)
        def _():
            tsl = pl.ds(NS * ROWS_PER_TILE, ROWS_REM)
            pltpu.sync_copy(z_hbm.at[tsl], acc.at[tsl])
            pltpu.sync_copy(z_hbm.at[pl.ds(N, NA - N)], acc.at[pl.ds(N, NA - N)])

        d0.wait()
        d1.wait()
        plsc.subcore_barrier()

        def body(r, carry):
            pltpu.async_copy(rows, acc.at[didx.at[r]], dsem, add=True)

            @pl.when(r >= DEG_K)
            def _():
                pltpu.make_async_copy(rows, acc.at[didx.at[0]], dsem).wait()

            return carry

        lax.fori_loop(0, CH_PER_W, body, 0)
        for _ in range(DEG_K):
            pltpu.make_async_copy(rows, acc.at[didx.at[0]], dsem).wait()

        plsc.subcore_barrier()
        _tile_rowwise_copy(s, acc, out_hbm.at[c])

    return k(dst2, ones_rows, zeros_acc)


def _sc_agg(g, src2, dst2, zeros_acc, F):
    @functools.partial(
        pl.kernel,
        out_type=jax.ShapeDtypeStruct((NC, N, F), jnp.float32),
        mesh=_mesh(),
        scratch_types=[
            pltpu.VMEM((AGG_HCH, CHUNK), jnp.int32),
            pltpu.VMEM((AGG_HCH, CHUNK), jnp.int32),
            pltpu.VMEM((AGG_K, CHUNK, F), jnp.float32),
            pltpu.VMEM_SHARED((N, F), jnp.float32),
            pltpu.VMEM_SHARED((NA, F), jnp.float32),
            pltpu.SemaphoreType.DMA((AGG_K,)),
            pltpu.SemaphoreType.DMA((AGG_K,)),
        ],
        compiler_params=pltpu.CompilerParams(use_tc_tiling_on_sc=False),
    )
    def k(g_hbm, src_hbm, dst_hbm, z_hbm, out_hbm, sidx, didx, rows, tbl, acc,
          gsem, ssem):
        c, s, w = _worker_ids()
        base = w * CH_PER_W
        rsl = pl.ds(s * ROWS_PER_TILE, ROWS_PER_TILE)
        d0 = pltpu.async_copy(z_hbm.at[rsl], acc.at[rsl], ssem.at[0])
        d1 = pltpu.async_copy(g_hbm.at[rsl], tbl.at[rsl], ssem.at[1])
        d2 = pltpu.async_copy(src_hbm.at[pl.ds(base, CH_PER_W)], sidx,
                              gsem.at[0])
        d3 = pltpu.async_copy(dst_hbm.at[pl.ds(base, CH_PER_W)], didx,
                              gsem.at[1])

        @pl.when(s == NS - 1)
        def _():
            tsl = pl.ds(NS * ROWS_PER_TILE, ROWS_REM)
            pltpu.sync_copy(z_hbm.at[tsl], acc.at[tsl])
            pltpu.sync_copy(g_hbm.at[tsl], tbl.at[tsl])
            pltpu.sync_copy(z_hbm.at[pl.ds(N, NA - N)], acc.at[pl.ds(N, NA - N)])

        d0.wait()
        d1.wait()
        d2.wait()
        d3.wait()
        plsc.subcore_barrier()

        n_groups = CH_PER_W // AGG_K

        def body(r, carry):
            for b in range(AGG_K):
                j = r * AGG_K + b
                pltpu.make_async_copy(
                    tbl.at[sidx.at[j]], rows.at[b], gsem.at[b]).wait()
                pltpu.async_copy(
                    rows.at[b], acc.at[didx.at[j]], ssem.at[b], add=True)

            @pl.when(r < n_groups - 1)
            def _():
                for b in range(AGG_K):
                    j = r * AGG_K + b
                    pltpu.make_async_copy(
                        rows.at[b], acc.at[didx.at[j]], ssem.at[b]).wait()
                    pltpu.async_copy(
                        tbl.at[sidx.at[j + AGG_K]], rows.at[b], gsem.at[b])

            return carry

        for b in range(AGG_K):
            pltpu.async_copy(tbl.at[sidx.at[b]], rows.at[b], gsem.at[b])
        lax.fori_loop(0, n_groups, body, 0)
        for b in range(AGG_K):
            pltpu.make_async_copy(
                rows.at[b], acc.at[didx.at[b]], ssem.at[b]).wait()

        plsc.subcore_barrier()
        _tile_rowwise_copy(s, acc, out_hbm.at[c])

    return k(g, src2, dst2, zeros_acc)


def _tc1_body(x_ref, w1_ref, degp_ref, g1_ref, dinv_ref):
    d = 1.0 + degp_ref[0] + degp_ref[1]
    dinv = lax.rsqrt(d)
    h = jnp.dot(x_ref[...], w1_ref[...], preferred_element_type=jnp.float32)
    g1_ref[...] = h * dinv[:, 0:1]
    dinv_ref[...] = dinv


def _tc2_body(p1_ref, g1_ref, dinv_ref, w2_ref, b1_ref, g2_ref):
    dinv = dinv_ref[...][:, 0:1]
    agg = p1_ref[0] + p1_ref[1] + g1_ref[...]
    h1 = jnp.maximum(dinv * agg + b1_ref[...], 0.0)
    h2 = jnp.dot(h1, w2_ref[...], preferred_element_type=jnp.float32)
    g2_ref[...] = h2 * dinv


def _tc3_body(p2_ref, g2_ref, dinv_ref, b2_ref, lw1_ref, lb1_ref, lw2_ref,
              lb2_ref, out_ref):
    dinv = dinv_ref[...][:, 0:1]
    agg = p2_ref[0] + p2_ref[1] + g2_ref[...]
    h2 = jnp.maximum(dinv * agg + b2_ref[...], 0.0)
    u = jnp.maximum(
        jnp.dot(h2, lw1_ref[...], preferred_element_type=jnp.float32)
        + lb1_ref[...], 0.0)
    out_ref[...] = (
        jnp.dot(u, lw2_ref[...], preferred_element_type=jnp.float32)
        + lb2_ref[...])


def kernel(x, edge_index, W1, b1, W2, b2, LW1, Lb1, LW2, Lb2):
    src = edge_index[0].astype(jnp.int32)
    dst = edge_index[1].astype(jnp.int32)
    pad = E_PAD - E
    src2 = jnp.concatenate([src, jnp.zeros((pad,), jnp.int32)])
    src2 = src2.reshape(NCHUNKS_PAD, CHUNK)
    dst2 = jnp.concatenate([dst, jnp.full((pad,), N, jnp.int32)])
    dst2 = dst2.reshape(NCHUNKS_PAD, CHUNK)
    ones_rows = jnp.ones((CHUNK, DEG_F), jnp.float32)
    zdeg = jnp.zeros((NA, DEG_F), jnp.float32)
    z16 = jnp.zeros((NA, HID), jnp.float32)
    z64 = jnp.zeros((NA, OUT_CH), jnp.float32)

    degp = _sc_deg(dst2, ones_rows, zdeg)

    g1, dinv = pl.pallas_call(
        _tc1_body,
        out_shape=[jax.ShapeDtypeStruct((N, HID), jnp.float32),
                   jax.ShapeDtypeStruct((N, DEG_F), jnp.float32)],
    )(x, W1, degp)

    p1 = _sc_agg(g1, src2, dst2, z16, HID)

    g2 = pl.pallas_call(
        _tc2_body,
        out_shape=jax.ShapeDtypeStruct((N, OUT_CH), jnp.float32),
    )(p1, g1, dinv, W2, b1.reshape(1, HID))

    p2 = _sc_agg(g2, src2, dst2, z64, OUT_CH)

    out = pl.pallas_call(
        _tc3_body,
        out_shape=jax.ShapeDtypeStruct((N, 1), jnp.float32),
    )(p2, g2, dinv, b2.reshape(1, OUT_CH), LW1, Lb1.reshape(1, OUT_CH),
      LW2, Lb2.reshape(1, 1))

    return out.reshape(N)
